# baseline (speedup 1.0000x reference)
import jax
import jax.numpy as jnp
from jax.experimental import pallas as pl
from jax.experimental.pallas import tpu as pltpu

_TOP_K = 2
_SCALE = 2.5
_NUM_EXPERTS = 8
_BLOCK_T = 1024
_NBUF = 3
_NSPLIT = 4


def _gate_kernel(hs_hbm, wt_ref, idx_ref, w_ref, buf, sems):
    n = hs_hbm.shape[0]
    nchunk = n // _BLOCK_T

    sub = _BLOCK_T // _NSPLIT

    def copies(slot, chunk):
        return [
            pltpu.make_async_copy(
                hs_hbm.at[pl.ds(chunk * _BLOCK_T + j * sub, sub), :],
                buf.at[slot, pl.ds(j * sub, sub), :],
                sems.at[slot, j],
            )
            for j in range(_NSPLIT)
        ]

    for k in range(_NBUF):
        for c in copies(k, k):
            c.start()

    for i in range(nchunk):
        slot = i % _NBUF
        for c in copies(slot, i):
            c.wait()
        hs = buf[slot]
        s = jnp.sum(hs[:, :128] * wt_ref[:128, 0], axis=1, keepdims=True)
        idx_ref[pl.ds(i * _BLOCK_T, _BLOCK_T), :] = jnp.concatenate([s, s], axis=1).astype(jnp.int32)
        w_ref[pl.ds(i * _BLOCK_T, _BLOCK_T), :] = jnp.concatenate([s, s], axis=1)
        if i + _NBUF < nchunk:
            for c in copies(slot, i + _NBUF):
                c.start()


def kernel(hidden_states, weight):
    bsz, seq_len, h = hidden_states.shape
    n = bsz * seq_len
    hs = hidden_states.reshape(n, h).astype(jnp.float32)
    wt = weight.astype(jnp.float32).T
    idx, w = pl.pallas_call(
        _gate_kernel,
        in_specs=[
            pl.BlockSpec(memory_space=pltpu.HBM),
            pl.BlockSpec(memory_space=pltpu.VMEM),
        ],
        out_specs=[
            pl.BlockSpec(memory_space=pltpu.VMEM),
            pl.BlockSpec(memory_space=pltpu.VMEM),
        ],
        out_shape=[
            jax.ShapeDtypeStruct((n, _TOP_K), jnp.int32),
            jax.ShapeDtypeStruct((n, _TOP_K), jnp.float32),
        ],
        scratch_shapes=[
            pltpu.VMEM((_NBUF, _BLOCK_T, h), jnp.float32),
            pltpu.SemaphoreType.DMA((_NBUF, _NSPLIT)),
        ],
    )(hs, wt)
    return idx, w


# speedup vs baseline: 1.3936x; 1.3936x over previous
import jax
import jax.numpy as jnp
from jax import lax
from jax.experimental import pallas as pl
from jax.experimental.pallas import tpu as pltpu

_TOP_K = 2
_SCALE = 2.5
_NUM_EXPERTS = 8
_BLOCK_T = 1024


def _gate_kernel(hs_ref, w_ref, idx_ref, wt_ref):
    hs = hs_ref[...]
    w8 = w_ref[...]
    logits = lax.dot_general(
        w8, hs, (((1,), (1,)), ((), ())),
        preferred_element_type=jnp.float32,
    )
    scores = jax.nn.sigmoid(logits)
    e = lax.broadcasted_iota(jnp.int32, scores.shape, 0)
    m1 = jnp.max(scores, axis=0, keepdims=True)
    i1 = jnp.min(jnp.where(scores == m1, e, _NUM_EXPERTS), axis=0, keepdims=True)
    masked = jnp.where(e == i1, -jnp.inf, scores)
    m2 = jnp.max(masked, axis=0, keepdims=True)
    i2 = jnp.min(jnp.where(masked == m2, e, _NUM_EXPERTS), axis=0, keepdims=True)
    denom = m1 + m2 + 1e-20
    idx_ref[...] = jnp.concatenate([i1, i2], axis=0)
    wt_ref[...] = jnp.concatenate([m1, m2], axis=0) * (_SCALE / denom)


def kernel(hidden_states, weight):
    bsz, seq_len, h = hidden_states.shape
    n = bsz * seq_len
    hs = hidden_states.reshape(n, h).astype(jnp.float32)
    w8 = weight.astype(jnp.float32)
    grid = (n // _BLOCK_T,)
    idx_t, w_t = pl.pallas_call(
        _gate_kernel,
        grid=grid,
        in_specs=[
            pl.BlockSpec((_BLOCK_T, h), lambda i: (i, 0)),
            pl.BlockSpec((_NUM_EXPERTS, h), lambda i: (0, 0)),
        ],
        out_specs=[
            pl.BlockSpec((_TOP_K, _BLOCK_T), lambda i: (0, i)),
            pl.BlockSpec((_TOP_K, _BLOCK_T), lambda i: (0, i)),
        ],
        out_shape=[
            jax.ShapeDtypeStruct((_TOP_K, n), jnp.int32),
            jax.ShapeDtypeStruct((_TOP_K, n), jnp.float32),
        ],
        compiler_params=pltpu.CompilerParams(
            dimension_semantics=("parallel",),
        ),
    )(hs, w8)
    return idx_t.T, w_t.T
